# Initial kernel scaffold; baseline (speedup 1.0000x reference)
import functools

import jax
import jax.numpy as jnp
from jax import lax
from jax.experimental import pallas as pl
from jax.experimental.pallas import tpu as pltpu
from jax.experimental.pallas import tpu_sc as plsc

_N = 10000
_E = 160000
_S = 8
_CONV = 64
_MUL = 64
_OUT = 128
_HID = _CONV + _MUL


_BE = 6400


def _ct_body(cp_ref,
             w1a_ref, b1a_ref, w2a_ref, b2a_ref, w3a_ref, b3a_ref,
             w4aa_ref, w4ba_ref, b4a_ref,
             w1b_ref, b1b_ref, w2b_ref, b2b_ref, w3b_ref, b3b_ref,
             w4ab_ref, w4bb_ref, b4b_ref,
             ct0_ref, ct1_ref):
    cp = cp_ref[...]

    def one(w1, b1, w2, b2, w3, b3, w4a, w4b, b4):
        o1 = jax.nn.sigmoid(jnp.dot(w1, cp, preferred_element_type=jnp.float32) + b1)
        o2 = jax.nn.sigmoid(jnp.dot(w2, cp, preferred_element_type=jnp.float32) + b2)
        o3 = jax.nn.sigmoid(jnp.dot(w3, cp, preferred_element_type=jnp.float32) + b3)
        g = o2 * o3
        pre = (jnp.dot(w4a, o1, preferred_element_type=jnp.float32)
               + jnp.dot(w4b, g, preferred_element_type=jnp.float32) + b4)
        return jnp.maximum(pre, 0.0)

    ct0_ref[...] = one(w1a_ref[...], b1a_ref[...], w2a_ref[...], b2a_ref[...],
                       w3a_ref[...], b3a_ref[...], w4aa_ref[...], w4ba_ref[...],
                       b4a_ref[...])
    ct1_ref[...] = one(w1b_ref[...], b1b_ref[...], w2b_ref[...], b2b_ref[...],
                       w3b_ref[...], b3b_ref[...], w4ab_ref[...], w4bb_ref[...],
                       b4b_ref[...])


def _ct_call(cpT, p0, p1):
    wspec = lambda r, c: pl.BlockSpec((r, c), lambda i: (0, 0))
    espec = pl.BlockSpec((_S, _BE), lambda i: (0, i))
    args = []
    for p in (p0, p1):
        (w1, b1, w2, b2, w3, b3, w4, b4) = p
        args += [w1, b1.reshape(_S, 1), w2, b2.reshape(_S, 1),
                 w3, b3.reshape(_S, 1), w4[:, :_S], w4[:, _S:],
                 b4.reshape(_S, 1)]
    in_specs = [espec]
    for _ in range(2):
        in_specs += [wspec(_S, _S), wspec(_S, 1), wspec(_S, _S), wspec(_S, 1),
                     wspec(_S, _S), wspec(_S, 1), wspec(_S, _S), wspec(_S, _S),
                     wspec(_S, 1)]
    return pl.pallas_call(
        _ct_body,
        grid=(_E // _BE,),
        in_specs=in_specs,
        out_specs=[espec, espec],
        out_shape=[jax.ShapeDtypeStruct((_S, _E), jnp.float32)] * 2,
    )(cpT, *args)


_BN = 2000


def _dense_body(x_ref, wf_ref, w5_ref, b5_ref, w6_ref, b6_ref, xw_ref, nm_ref):
    x = x_ref[...]
    xw_ref[...] = jnp.dot(x, wf_ref[...], preferred_element_type=jnp.float32)
    m5 = jnp.dot(x, w5_ref[...], preferred_element_type=jnp.float32) + b5_ref[...]
    m6 = jnp.dot(x, w6_ref[...], preferred_element_type=jnp.float32) + b6_ref[...]
    nm_ref[...] = m5 * m6


def _dense_call(x, Wf, w5T, b5, w6T, b6):
    cin = x.shape[1]
    return pl.pallas_call(
        _dense_body,
        grid=(_N // _BN,),
        in_specs=[
            pl.BlockSpec((_BN, cin), lambda i: (i, 0)),
            pl.BlockSpec((cin, _S * _CONV), lambda i: (0, 0)),
            pl.BlockSpec((cin, _MUL), lambda i: (0, 0)),
            pl.BlockSpec((1, _MUL), lambda i: (0, 0)),
            pl.BlockSpec((cin, _MUL), lambda i: (0, 0)),
            pl.BlockSpec((1, _MUL), lambda i: (0, 0)),
        ],
        out_specs=[
            pl.BlockSpec((_BN, _S * _CONV), lambda i: (i, 0)),
            pl.BlockSpec((_BN, _MUL), lambda i: (i, 0)),
        ],
        out_shape=[
            jax.ShapeDtypeStruct((_N, _S * _CONV), jnp.float32),
            jax.ShapeDtypeStruct((_N, _MUL), jnp.float32),
        ],
    )(x, Wf, w5T, b5.reshape(1, _MUL), w6T, b6.reshape(1, _MUL))


def _comb_dense_body(p0_ref, p1_ref, nm_ref, wf_ref, w5_ref, b5_ref,
                     w6_ref, b6_ref, xw_ref, nm2_ref):
    aggr = jnp.maximum(p0_ref[...] + p1_ref[...], 0.0)
    nm = jnp.maximum(nm_ref[...], 0.0)
    h = jnp.concatenate([aggr, nm], axis=1)
    xw_ref[...] = jnp.dot(h, wf_ref[...], preferred_element_type=jnp.float32)
    m5 = jnp.dot(h, w5_ref[...], preferred_element_type=jnp.float32) + b5_ref[...]
    m6 = jnp.dot(h, w6_ref[...], preferred_element_type=jnp.float32) + b6_ref[...]
    nm2_ref[...] = m5 * m6


def _comb_dense_call(partials, nm, Wf, w5T, b5, w6T, b6):
    nblk = _N // _BN
    return pl.pallas_call(
        _comb_dense_body,
        grid=(nblk,),
        in_specs=[
            pl.BlockSpec((_BN, _CONV), lambda i: (i, 0)),
            pl.BlockSpec((_BN, _CONV), lambda i: (i + nblk, 0)),
            pl.BlockSpec((_BN, _MUL), lambda i: (i, 0)),
            pl.BlockSpec((_HID, _S * _CONV), lambda i: (0, 0)),
            pl.BlockSpec((_HID, _MUL), lambda i: (0, 0)),
            pl.BlockSpec((1, _MUL), lambda i: (0, 0)),
            pl.BlockSpec((_HID, _MUL), lambda i: (0, 0)),
            pl.BlockSpec((1, _MUL), lambda i: (0, 0)),
        ],
        out_specs=[
            pl.BlockSpec((_BN, _S * _CONV), lambda i: (i, 0)),
            pl.BlockSpec((_BN, _MUL), lambda i: (i, 0)),
        ],
        out_shape=[
            jax.ShapeDtypeStruct((_N, _S * _CONV), jnp.float32),
            jax.ShapeDtypeStruct((_N, _MUL), jnp.float32),
        ],
    )(partials, partials, nm, Wf, w5T, b5.reshape(1, _MUL), w6T,
      b6.reshape(1, _MUL))


def _comb_ro_body(p0_ref, p1_ref, nm_ref, ro_ref, rob_ref, out_ref):
    aggr = jnp.maximum(p0_ref[...] + p1_ref[...], 0.0)
    nm = jnp.maximum(nm_ref[...], 0.0)
    h = jnp.concatenate([aggr, nm], axis=1)
    out_ref[...] = (jnp.dot(h, ro_ref[...], preferred_element_type=jnp.float32)
                    + rob_ref[...])


def _comb_ro_call(partials, nm, roT, rob):
    nblk = _N // _BN
    return pl.pallas_call(
        _comb_ro_body,
        grid=(nblk,),
        in_specs=[
            pl.BlockSpec((_BN, _CONV), lambda i: (i, 0)),
            pl.BlockSpec((_BN, _CONV), lambda i: (i + nblk, 0)),
            pl.BlockSpec((_BN, _MUL), lambda i: (i, 0)),
            pl.BlockSpec((_HID, _OUT), lambda i: (0, 0)),
            pl.BlockSpec((1, _OUT), lambda i: (0, 0)),
        ],
        out_specs=pl.BlockSpec((_BN, _OUT), lambda i: (i, 0)),
        out_shape=jax.ShapeDtypeStruct((_N, _OUT), jnp.float32),
    )(partials, partials, nm, roT, rob.reshape(1, _OUT))


_NW = 32
_EPW = _E // _NW
_SC_CHUNK = 40
_NCHUNK = _EPW // _SC_CHUNK
_NPS = _N // 16


def _sc_edge_body(xw_hbm, ct_hbm, src_hbm, dst_hbm, zeros_hbm, out_hbm,
                  srcv, dstv, ctv, rows, msg, aggr_sh, sem):
    cid = lax.axis_index("c")
    sid = lax.axis_index("s")
    wid = sid * 2 + cid
    base0 = wid * _EPW

    pltpu.sync_copy(zeros_hbm.at[pl.ds(sid * _NPS, _NPS)],
                    aggr_sh.at[pl.ds(sid * _NPS, _NPS)])
    plsc.subcore_barrier()

    def chunk_body(k, carry):
        base = base0 + k * _SC_CHUNK
        pltpu.sync_copy(src_hbm.at[pl.ds(base, _SC_CHUNK)], srcv)
        pltpu.sync_copy(dst_hbm.at[pl.ds(base, _SC_CHUNK)], dstv)
        pltpu.sync_copy(ct_hbm.at[:, pl.ds(base, _SC_CHUNK)], ctv)
        pltpu.async_copy(xw_hbm.at[srcv], rows, sem).wait()

        def edge_body(e, c2):
            coef = [jnp.full((16,), ctv[s, e], jnp.float32) for s in range(_S)]
            for cb in range(_CONV // 16):
                acc = rows[e, pl.ds(cb * 16, 16)] * coef[0]
                for s in range(1, _S):
                    acc = acc + rows[e, pl.ds(s * _CONV + cb * 16, 16)] * coef[s]
                msg[e, pl.ds(cb * 16, 16)] = acc
            return c2

        lax.fori_loop(0, _SC_CHUNK, edge_body, 0)
        pltpu.sync_copy(msg, aggr_sh.at[dstv], add=True)
        return carry

    lax.fori_loop(0, _NCHUNK, chunk_body, 0)
    plsc.subcore_barrier()
    pltpu.sync_copy(aggr_sh.at[pl.ds(sid * _NPS, _NPS)],
                    out_hbm.at[pl.ds(cid * _N + sid * _NPS, _NPS)])


_sc_edge_call = functools.partial(
    pl.kernel,
    out_type=jax.ShapeDtypeStruct((2 * _N, _CONV), jnp.float32),
    mesh=plsc.VectorSubcoreMesh(core_axis_name="c", subcore_axis_name="s"),
    scratch_types=[
        pltpu.VMEM((_SC_CHUNK,), jnp.int32),
        pltpu.VMEM((_SC_CHUNK,), jnp.int32),
        pltpu.VMEM((_S, _SC_CHUNK), jnp.float32),
        pltpu.VMEM((_SC_CHUNK, _S * _CONV), jnp.float32),
        pltpu.VMEM((_SC_CHUNK, _CONV), jnp.float32),
        pltpu.VMEM_SHARED((_N, _CONV), jnp.float32),
        pltpu.SemaphoreType.DMA,
    ],
)(_sc_edge_body)


def kernel(x, edge_index, C_prime,
           l0_w1, l0_b1, l0_w2, l0_b2, l0_w3, l0_b3, l0_w4, l0_b4, l0_W,
           l0_w5, l0_b5, l0_w6, l0_b6,
           l1_w1, l1_b1, l1_w2, l1_b2, l1_w3, l1_b3, l1_w4, l1_b4, l1_W,
           l1_w5, l1_b5, l1_w6, l1_b6,
           ro_w, ro_b):
    ei = edge_index.astype(jnp.int32)
    src = ei[0]
    dst = ei[1]
    cpT = C_prime.T

    ct0T, ct1T = _ct_call(
        cpT,
        (l0_w1, l0_b1, l0_w2, l0_b2, l0_w3, l0_b3, l0_w4, l0_b4),
        (l1_w1, l1_b1, l1_w2, l1_b2, l1_w3, l1_b3, l1_w4, l1_b4))

    W0f = l0_W.transpose(1, 0, 2).reshape(-1, _S * _CONV)
    W1f = l1_W.transpose(1, 0, 2).reshape(-1, _S * _CONV)
    zeros = jnp.zeros((_N, _CONV), jnp.float32)

    xw1, nm1 = _dense_call(x, W0f, l0_w5.T, l0_b5, l0_w6.T, l0_b6)
    part1 = _sc_edge_call(xw1, ct0T, src, dst, zeros)
    xw2, nm2 = _comb_dense_call(part1, nm1, W1f, l1_w5.T, l1_b5, l1_w6.T, l1_b6)
    part2 = _sc_edge_call(xw2, ct1T, src, dst, zeros)
    return _comb_ro_call(part2, nm2, ro_w.T, ro_b)


# speedup vs baseline: 3.2446x; 3.2446x over previous
import functools

import jax
import jax.numpy as jnp
from jax import lax
from jax.experimental import pallas as pl
from jax.experimental.pallas import tpu as pltpu
from jax.experimental.pallas import tpu_sc as plsc

_N = 10000
_NP = 10240
_E = 160000
_S = 8
_CONV = 64
_MUL = 64
_OUT = 128
_HID = _CONV + _MUL


_BE = 6400


def _ct_body(cp_ref,
             w1a_ref, b1a_ref, w2a_ref, b2a_ref, w3a_ref, b3a_ref,
             w4aa_ref, w4ba_ref, b4a_ref,
             w1b_ref, b1b_ref, w2b_ref, b2b_ref, w3b_ref, b3b_ref,
             w4ab_ref, w4bb_ref, b4b_ref,
             ct0_ref, ct1_ref):
    cp = cp_ref[...]

    def one(w1, b1, w2, b2, w3, b3, w4a, w4b, b4):
        o1 = jax.nn.sigmoid(jnp.dot(w1, cp, preferred_element_type=jnp.float32) + b1)
        o2 = jax.nn.sigmoid(jnp.dot(w2, cp, preferred_element_type=jnp.float32) + b2)
        o3 = jax.nn.sigmoid(jnp.dot(w3, cp, preferred_element_type=jnp.float32) + b3)
        g = o2 * o3
        pre = (jnp.dot(w4a, o1, preferred_element_type=jnp.float32)
               + jnp.dot(w4b, g, preferred_element_type=jnp.float32) + b4)
        return jnp.maximum(pre, 0.0)

    ct0_ref[...] = one(w1a_ref[...], b1a_ref[...], w2a_ref[...], b2a_ref[...],
                       w3a_ref[...], b3a_ref[...], w4aa_ref[...], w4ba_ref[...],
                       b4a_ref[...])
    ct1_ref[...] = one(w1b_ref[...], b1b_ref[...], w2b_ref[...], b2b_ref[...],
                       w3b_ref[...], b3b_ref[...], w4ab_ref[...], w4bb_ref[...],
                       b4b_ref[...])


def _ct_call(cpT, p0, p1):
    wspec = lambda r, c: pl.BlockSpec((r, c), lambda i: (0, 0))
    espec = pl.BlockSpec((_S, _BE), lambda i: (0, i))
    args = []
    for p in (p0, p1):
        (w1, b1, w2, b2, w3, b3, w4, b4) = p
        args += [w1, b1.reshape(_S, 1), w2, b2.reshape(_S, 1),
                 w3, b3.reshape(_S, 1), w4[:, :_S], w4[:, _S:],
                 b4.reshape(_S, 1)]
    in_specs = [espec]
    for _ in range(2):
        in_specs += [wspec(_S, _S), wspec(_S, 1), wspec(_S, _S), wspec(_S, 1),
                     wspec(_S, _S), wspec(_S, 1), wspec(_S, _S), wspec(_S, _S),
                     wspec(_S, 1)]
    return pl.pallas_call(
        _ct_body,
        grid=(_E // _BE,),
        in_specs=in_specs,
        out_specs=[espec, espec],
        out_shape=[jax.ShapeDtypeStruct((_S, _E), jnp.float32)] * 2,
    )(cpT, *args)


_BN = 2048


def _dense_body(x_ref, wf_ref, w5_ref, b5_ref, w6_ref, b6_ref, xw_ref, nm_ref):
    x = x_ref[...]
    xw_ref[...] = jnp.dot(x, wf_ref[...], preferred_element_type=jnp.float32)
    m5 = jnp.dot(x, w5_ref[...], preferred_element_type=jnp.float32) + b5_ref[...]
    m6 = jnp.dot(x, w6_ref[...], preferred_element_type=jnp.float32) + b6_ref[...]
    nm_ref[...] = m5 * m6


def _dense_call(x, Wf, w5T, b5, w6T, b6):
    cin = x.shape[1]
    return pl.pallas_call(
        _dense_body,
        grid=(_NP // _BN,),
        in_specs=[
            pl.BlockSpec((_BN, cin), lambda i: (i, 0)),
            pl.BlockSpec((cin, _S * _CONV), lambda i: (0, 0)),
            pl.BlockSpec((cin, _MUL), lambda i: (0, 0)),
            pl.BlockSpec((1, _MUL), lambda i: (0, 0)),
            pl.BlockSpec((cin, _MUL), lambda i: (0, 0)),
            pl.BlockSpec((1, _MUL), lambda i: (0, 0)),
        ],
        out_specs=[
            pl.BlockSpec((_BN, _S * _CONV), lambda i: (i, 0)),
            pl.BlockSpec((_BN, _MUL), lambda i: (i, 0)),
        ],
        out_shape=[
            jax.ShapeDtypeStruct((_NP, _S * _CONV), jnp.float32),
            jax.ShapeDtypeStruct((_NP, _MUL), jnp.float32),
        ],
    )(x, Wf, w5T, b5.reshape(1, _MUL), w6T, b6.reshape(1, _MUL))


def _comb_dense_body(p0_ref, p1_ref, nm_ref, wf_ref, w5_ref, b5_ref,
                     w6_ref, b6_ref, xw_ref, nm2_ref):
    aggr = jnp.maximum(p0_ref[...] + p1_ref[...], 0.0)
    nm = jnp.maximum(nm_ref[...], 0.0)
    h = jnp.concatenate([aggr, nm], axis=1)
    xw_ref[...] = jnp.dot(h, wf_ref[...], preferred_element_type=jnp.float32)
    m5 = jnp.dot(h, w5_ref[...], preferred_element_type=jnp.float32) + b5_ref[...]
    m6 = jnp.dot(h, w6_ref[...], preferred_element_type=jnp.float32) + b6_ref[...]
    nm2_ref[...] = m5 * m6


def _comb_dense_call(partials, nm, Wf, w5T, b5, w6T, b6):
    nblk = _NP // _BN
    return pl.pallas_call(
        _comb_dense_body,
        grid=(nblk,),
        in_specs=[
            pl.BlockSpec((_BN, _CONV), lambda i: (i, 0)),
            pl.BlockSpec((_BN, _CONV), lambda i: (i + nblk, 0)),
            pl.BlockSpec((_BN, _MUL), lambda i: (i, 0)),
            pl.BlockSpec((_HID, _S * _CONV), lambda i: (0, 0)),
            pl.BlockSpec((_HID, _MUL), lambda i: (0, 0)),
            pl.BlockSpec((1, _MUL), lambda i: (0, 0)),
            pl.BlockSpec((_HID, _MUL), lambda i: (0, 0)),
            pl.BlockSpec((1, _MUL), lambda i: (0, 0)),
        ],
        out_specs=[
            pl.BlockSpec((_BN, _S * _CONV), lambda i: (i, 0)),
            pl.BlockSpec((_BN, _MUL), lambda i: (i, 0)),
        ],
        out_shape=[
            jax.ShapeDtypeStruct((_NP, _S * _CONV), jnp.float32),
            jax.ShapeDtypeStruct((_NP, _MUL), jnp.float32),
        ],
    )(partials, partials, nm, Wf, w5T, b5.reshape(1, _MUL), w6T,
      b6.reshape(1, _MUL))


def _comb_ro_body(p0_ref, p1_ref, nm_ref, ro_ref, rob_ref, out_ref):
    aggr = jnp.maximum(p0_ref[...] + p1_ref[...], 0.0)
    nm = jnp.maximum(nm_ref[...], 0.0)
    h = jnp.concatenate([aggr, nm], axis=1)
    out_ref[...] = (jnp.dot(h, ro_ref[...], preferred_element_type=jnp.float32)
                    + rob_ref[...])


def _comb_ro_call(partials, nm, roT, rob):
    nblk = _NP // _BN
    return pl.pallas_call(
        _comb_ro_body,
        grid=(nblk,),
        in_specs=[
            pl.BlockSpec((_BN, _CONV), lambda i: (i, 0)),
            pl.BlockSpec((_BN, _CONV), lambda i: (i + nblk, 0)),
            pl.BlockSpec((_BN, _MUL), lambda i: (i, 0)),
            pl.BlockSpec((_HID, _OUT), lambda i: (0, 0)),
            pl.BlockSpec((1, _OUT), lambda i: (0, 0)),
        ],
        out_specs=pl.BlockSpec((_BN, _OUT), lambda i: (i, 0)),
        out_shape=jax.ShapeDtypeStruct((_NP, _OUT), jnp.float32),
    )(partials, partials, nm, roT, rob.reshape(1, _OUT))


_NW = 32
_SC_CHUNK = 128
_NCHUNK = _E // _SC_CHUNK
_FULL_ROUNDS = _NCHUNK // _NW
_NPS = _NP // 16


def _bcast_lane(v, s):
    idx = jnp.full((16,), s, jnp.int32)
    dnums = lax.GatherDimensionNumbers(
        offset_dims=(), collapsed_slice_dims=(0,), start_index_map=(0,))
    return lax.gather(v, idx[:, None], dnums, (1,),
                      mode=lax.GatherScatterMode.PROMISE_IN_BOUNDS)


def _sc_edge_body(xw_hbm, ct_hbm, src_hbm, dst_hbm, zeros_hbm, out_hbm,
                  srcv, dstv, ctv, rows, msg, aggr_sh, sem):
    cid = lax.axis_index("c")
    sid = lax.axis_index("s")
    wid = sid * 2 + cid

    pltpu.sync_copy(zeros_hbm.at[pl.ds(sid * _NPS, _NPS)],
                    aggr_sh.at[pl.ds(sid * _NPS, _NPS)])
    plsc.subcore_barrier()

    def chunk_work(c):
        base = c * _SC_CHUNK
        pltpu.sync_copy(src_hbm.at[pl.ds(base, _SC_CHUNK)], srcv)
        pltpu.sync_copy(dst_hbm.at[pl.ds(base, _SC_CHUNK)], dstv)
        pltpu.sync_copy(ct_hbm.at[pl.ds(base * 16, _SC_CHUNK * 16)], ctv)
        pltpu.async_copy(xw_hbm.at[srcv], rows, sem).wait()

        def edge_body(e, c2):
            cvec = ctv[pl.ds(e * 16, 16)]
            coef = [_bcast_lane(cvec, s) for s in range(_S)]
            for cb in range(_CONV // 16):
                acc = rows[e, pl.ds(cb * 16, 16)] * coef[0]
                for s in range(1, _S):
                    acc = acc + rows[e, pl.ds(s * _CONV + cb * 16, 16)] * coef[s]
                msg[e, pl.ds(cb * 16, 16)] = acc
            return c2

        lax.fori_loop(0, _SC_CHUNK, edge_body, 0)
        pltpu.sync_copy(msg, aggr_sh.at[dstv], add=True)

    def chunk_body(k, carry):
        chunk_work(wid + k * _NW)
        return carry

    lax.fori_loop(0, _FULL_ROUNDS, chunk_body, 0)

    @pl.when(wid + _FULL_ROUNDS * _NW < _NCHUNK)
    def _():
        chunk_work(wid + _FULL_ROUNDS * _NW)

    plsc.subcore_barrier()
    pltpu.sync_copy(aggr_sh.at[pl.ds(sid * _NPS, _NPS)],
                    out_hbm.at[pl.ds(cid * _NP + sid * _NPS, _NPS)])


_sc_edge_call = functools.partial(
    pl.kernel,
    out_type=jax.ShapeDtypeStruct((2 * _NP, _CONV), jnp.float32),
    mesh=plsc.VectorSubcoreMesh(core_axis_name="c", subcore_axis_name="s"),
    compiler_params=pltpu.CompilerParams(use_tc_tiling_on_sc=False),
    scratch_types=[
        pltpu.VMEM((_SC_CHUNK,), jnp.int32),
        pltpu.VMEM((_SC_CHUNK,), jnp.int32),
        pltpu.VMEM((_SC_CHUNK * 16,), jnp.float32),
        pltpu.VMEM((_SC_CHUNK, _S * _CONV), jnp.float32),
        pltpu.VMEM((_SC_CHUNK, _CONV), jnp.float32),
        pltpu.VMEM_SHARED((_NP, _CONV), jnp.float32),
        pltpu.SemaphoreType.DMA,
    ],
)(_sc_edge_body)


def kernel(x, edge_index, C_prime,
           l0_w1, l0_b1, l0_w2, l0_b2, l0_w3, l0_b3, l0_w4, l0_b4, l0_W,
           l0_w5, l0_b5, l0_w6, l0_b6,
           l1_w1, l1_b1, l1_w2, l1_b2, l1_w3, l1_b3, l1_w4, l1_b4, l1_W,
           l1_w5, l1_b5, l1_w6, l1_b6,
           ro_w, ro_b):
    ei = edge_index.astype(jnp.int32)
    src = ei[0]
    dst = ei[1]
    cpT = C_prime.T
    xp = jnp.pad(x, ((0, _NP - _N), (0, 0)))

    ct0T, ct1T = _ct_call(
        cpT,
        (l0_w1, l0_b1, l0_w2, l0_b2, l0_w3, l0_b3, l0_w4, l0_b4),
        (l1_w1, l1_b1, l1_w2, l1_b2, l1_w3, l1_b3, l1_w4, l1_b4))
    ct0 = jnp.pad(ct0T, ((0, 8), (0, 0))).T.reshape(-1)
    ct1 = jnp.pad(ct1T, ((0, 8), (0, 0))).T.reshape(-1)

    W0f = l0_W.transpose(1, 0, 2).reshape(-1, _S * _CONV)
    W1f = l1_W.transpose(1, 0, 2).reshape(-1, _S * _CONV)
    zeros = jnp.zeros((_NP, _CONV), jnp.float32)

    xw1, nm1 = _dense_call(xp, W0f, l0_w5.T, l0_b5, l0_w6.T, l0_b6)
    part1 = _sc_edge_call(xw1, ct0, src, dst, zeros)
    xw2, nm2 = _comb_dense_call(part1, nm1, W1f, l1_w5.T, l1_b5, l1_w6.T, l1_b6)
    part2 = _sc_edge_call(xw2, ct1, src, dst, zeros)
    out = _comb_ro_call(part2, nm2, ro_w.T, ro_b)
    return out[:_N]


# speedup vs baseline: 4.5110x; 1.3903x over previous
import functools

import jax
import jax.numpy as jnp
from jax import lax
from jax.experimental import pallas as pl
from jax.experimental.pallas import tpu as pltpu
from jax.experimental.pallas import tpu_sc as plsc

_N = 10000
_NP = 10240
_E = 160000
_S = 8
_CONV = 64
_MUL = 64
_OUT = 128
_HID = _CONV + _MUL


_BE = 6400


def _ct_body(cp_ref,
             w1a_ref, b1a_ref, w2a_ref, b2a_ref, w3a_ref, b3a_ref,
             w4aa_ref, w4ba_ref, b4a_ref,
             w1b_ref, b1b_ref, w2b_ref, b2b_ref, w3b_ref, b3b_ref,
             w4ab_ref, w4bb_ref, b4b_ref,
             ct0_ref, ct1_ref):
    cp = cp_ref[...]

    def one(w1, b1, w2, b2, w3, b3, w4a, w4b, b4):
        o1 = jax.nn.sigmoid(jnp.dot(w1, cp, preferred_element_type=jnp.float32) + b1)
        o2 = jax.nn.sigmoid(jnp.dot(w2, cp, preferred_element_type=jnp.float32) + b2)
        o3 = jax.nn.sigmoid(jnp.dot(w3, cp, preferred_element_type=jnp.float32) + b3)
        g = o2 * o3
        pre = (jnp.dot(w4a, o1, preferred_element_type=jnp.float32)
               + jnp.dot(w4b, g, preferred_element_type=jnp.float32) + b4)
        return jnp.maximum(pre, 0.0)

    ct0_ref[...] = one(w1a_ref[...], b1a_ref[...], w2a_ref[...], b2a_ref[...],
                       w3a_ref[...], b3a_ref[...], w4aa_ref[...], w4ba_ref[...],
                       b4a_ref[...])
    ct1_ref[...] = one(w1b_ref[...], b1b_ref[...], w2b_ref[...], b2b_ref[...],
                       w3b_ref[...], b3b_ref[...], w4ab_ref[...], w4bb_ref[...],
                       b4b_ref[...])


def _ct_call(cpT, p0, p1):
    wspec = lambda r, c: pl.BlockSpec((r, c), lambda i: (0, 0))
    espec = pl.BlockSpec((_S, _BE), lambda i: (0, i))
    args = []
    for p in (p0, p1):
        (w1, b1, w2, b2, w3, b3, w4, b4) = p
        args += [w1, b1.reshape(_S, 1), w2, b2.reshape(_S, 1),
                 w3, b3.reshape(_S, 1), w4[:, :_S], w4[:, _S:],
                 b4.reshape(_S, 1)]
    in_specs = [espec]
    for _ in range(2):
        in_specs += [wspec(_S, _S), wspec(_S, 1), wspec(_S, _S), wspec(_S, 1),
                     wspec(_S, _S), wspec(_S, 1), wspec(_S, _S), wspec(_S, _S),
                     wspec(_S, 1)]
    return pl.pallas_call(
        _ct_body,
        grid=(_E // _BE,),
        in_specs=in_specs,
        out_specs=[espec, espec],
        out_shape=[jax.ShapeDtypeStruct((_S, _E), jnp.float32)] * 2,
    )(cpT, *args)


_BN = 2048


def _dense_body(x_ref, wf_ref, w5_ref, b5_ref, w6_ref, b6_ref, xw_ref, nm_ref):
    x = x_ref[...]
    xw_ref[...] = jnp.dot(x, wf_ref[...], preferred_element_type=jnp.float32)
    m5 = jnp.dot(x, w5_ref[...], preferred_element_type=jnp.float32) + b5_ref[...]
    m6 = jnp.dot(x, w6_ref[...], preferred_element_type=jnp.float32) + b6_ref[...]
    nm_ref[...] = m5 * m6


def _dense_call(x, Wf, w5T, b5, w6T, b6):
    cin = x.shape[1]
    return pl.pallas_call(
        _dense_body,
        grid=(_NP // _BN,),
        in_specs=[
            pl.BlockSpec((_BN, cin), lambda i: (i, 0)),
            pl.BlockSpec((cin, _S * _CONV), lambda i: (0, 0)),
            pl.BlockSpec((cin, _MUL), lambda i: (0, 0)),
            pl.BlockSpec((1, _MUL), lambda i: (0, 0)),
            pl.BlockSpec((cin, _MUL), lambda i: (0, 0)),
            pl.BlockSpec((1, _MUL), lambda i: (0, 0)),
        ],
        out_specs=[
            pl.BlockSpec((_BN, _S * _CONV), lambda i: (i, 0)),
            pl.BlockSpec((_BN, _MUL), lambda i: (i, 0)),
        ],
        out_shape=[
            jax.ShapeDtypeStruct((_NP, _S * _CONV), jnp.float32),
            jax.ShapeDtypeStruct((_NP, _MUL), jnp.float32),
        ],
    )(x, Wf, w5T, b5.reshape(1, _MUL), w6T, b6.reshape(1, _MUL))


def _comb_dense_body(p0_ref, p1_ref, nm_ref, wf_ref, w5_ref, b5_ref,
                     w6_ref, b6_ref, xw_ref, nm2_ref):
    aggr = jnp.maximum(p0_ref[...] + p1_ref[...], 0.0)
    nm = jnp.maximum(nm_ref[...], 0.0)
    h = jnp.concatenate([aggr, nm], axis=1)
    xw_ref[...] = jnp.dot(h, wf_ref[...], preferred_element_type=jnp.float32)
    m5 = jnp.dot(h, w5_ref[...], preferred_element_type=jnp.float32) + b5_ref[...]
    m6 = jnp.dot(h, w6_ref[...], preferred_element_type=jnp.float32) + b6_ref[...]
    nm2_ref[...] = m5 * m6


def _comb_dense_call(partials, nm, Wf, w5T, b5, w6T, b6):
    nblk = _NP // _BN
    return pl.pallas_call(
        _comb_dense_body,
        grid=(nblk,),
        in_specs=[
            pl.BlockSpec((_BN, _CONV), lambda i: (i, 0)),
            pl.BlockSpec((_BN, _CONV), lambda i: (i + nblk, 0)),
            pl.BlockSpec((_BN, _MUL), lambda i: (i, 0)),
            pl.BlockSpec((_HID, _S * _CONV), lambda i: (0, 0)),
            pl.BlockSpec((_HID, _MUL), lambda i: (0, 0)),
            pl.BlockSpec((1, _MUL), lambda i: (0, 0)),
            pl.BlockSpec((_HID, _MUL), lambda i: (0, 0)),
            pl.BlockSpec((1, _MUL), lambda i: (0, 0)),
        ],
        out_specs=[
            pl.BlockSpec((_BN, _S * _CONV), lambda i: (i, 0)),
            pl.BlockSpec((_BN, _MUL), lambda i: (i, 0)),
        ],
        out_shape=[
            jax.ShapeDtypeStruct((_NP, _S * _CONV), jnp.float32),
            jax.ShapeDtypeStruct((_NP, _MUL), jnp.float32),
        ],
    )(partials, partials, nm, Wf, w5T, b5.reshape(1, _MUL), w6T,
      b6.reshape(1, _MUL))


def _comb_ro_body(p0_ref, p1_ref, nm_ref, ro_ref, rob_ref, out_ref):
    aggr = jnp.maximum(p0_ref[...] + p1_ref[...], 0.0)
    nm = jnp.maximum(nm_ref[...], 0.0)
    h = jnp.concatenate([aggr, nm], axis=1)
    out_ref[...] = (jnp.dot(h, ro_ref[...], preferred_element_type=jnp.float32)
                    + rob_ref[...])


def _comb_ro_call(partials, nm, roT, rob):
    nblk = _NP // _BN
    return pl.pallas_call(
        _comb_ro_body,
        grid=(nblk,),
        in_specs=[
            pl.BlockSpec((_BN, _CONV), lambda i: (i, 0)),
            pl.BlockSpec((_BN, _CONV), lambda i: (i + nblk, 0)),
            pl.BlockSpec((_BN, _MUL), lambda i: (i, 0)),
            pl.BlockSpec((_HID, _OUT), lambda i: (0, 0)),
            pl.BlockSpec((1, _OUT), lambda i: (0, 0)),
        ],
        out_specs=pl.BlockSpec((_BN, _OUT), lambda i: (i, 0)),
        out_shape=jax.ShapeDtypeStruct((_NP, _OUT), jnp.float32),
    )(partials, partials, nm, roT, rob.reshape(1, _OUT))


_NW = 32
_SC_CHUNK = 64
_NCHUNK = _E // _SC_CHUNK
_BASE_K = _NCHUNK // _NW
_EXTRA = _NCHUNK - _BASE_K * _NW
_STAGE = (_BASE_K + 1) * _SC_CHUNK
_NPS = _NP // 16


def _bcast_lane(v, s):
    idx = jnp.full((16,), s, jnp.int32)
    dnums = lax.GatherDimensionNumbers(
        offset_dims=(), collapsed_slice_dims=(0,), start_index_map=(0,))
    return lax.gather(v, idx[:, None], dnums, (1,),
                      mode=lax.GatherScatterMode.PROMISE_IN_BOUNDS)


def _sc_edge_body(xw_hbm, ct_hbm, src_hbm, dst_hbm, zeros_hbm, out_hbm,
                  src_all, dstv0, dstv1, ctv0, ctv1, rows0, rows1, msg,
                  aggr_sh, sem0, sem1):
    cid = lax.axis_index("c")
    sid = lax.axis_index("s")
    wid = sid * 2 + cid
    nk = jnp.where(wid < _EXTRA, _BASE_K + 1, _BASE_K)
    base_e = (wid * _BASE_K + jnp.minimum(wid, _EXTRA)) * _SC_CHUNK

    pltpu.sync_copy(zeros_hbm.at[pl.ds(sid * _NPS, _NPS)],
                    aggr_sh.at[pl.ds(sid * _NPS, _NPS)])
    pltpu.sync_copy(src_hbm.at[pl.ds(base_e, _STAGE)], src_all)
    plsc.subcore_barrier()

    bufs = ((dstv0, ctv0, rows0, sem0), (dstv1, ctv1, rows1, sem1))

    def enqueue(j, b):
        dstv, ctv, rows, sem = bufs[b]
        pltpu.async_copy(xw_hbm.at[src_all.at[pl.ds(j * _SC_CHUNK, _SC_CHUNK)]],
                         rows, sem)
        pltpu.async_copy(
            ct_hbm.at[pl.ds((base_e + j * _SC_CHUNK) * 16, _SC_CHUNK * 16)],
            ctv, sem)
        pltpu.async_copy(dst_hbm.at[pl.ds(base_e + j * _SC_CHUNK, _SC_CHUNK)],
                         dstv, sem)

    def process(j, b):
        dstv, ctv, rows, sem = bufs[b]
        pltpu.make_async_copy(
            xw_hbm.at[src_all.at[pl.ds(j * _SC_CHUNK, _SC_CHUNK)]],
            rows, sem).wait()
        pltpu.make_async_copy(
            ct_hbm.at[pl.ds((base_e + j * _SC_CHUNK) * 16, _SC_CHUNK * 16)],
            ctv, sem).wait()
        pltpu.make_async_copy(
            dst_hbm.at[pl.ds(base_e + j * _SC_CHUNK, _SC_CHUNK)],
            dstv, sem).wait()

        def edge_body(e, c2):
            cvec = ctv[pl.ds(e * 16, 16)]
            coef = [_bcast_lane(cvec, s) for s in range(_S)]
            for cb in range(_CONV // 16):
                acc = rows[e, pl.ds(cb * 16, 16)] * coef[0]
                for s in range(1, _S):
                    acc = acc + rows[e, pl.ds(s * _CONV + cb * 16, 16)] * coef[s]
                msg[e, pl.ds(cb * 16, 16)] = acc
            return c2

        lax.fori_loop(0, _SC_CHUNK, edge_body, 0, unroll=4)
        pltpu.sync_copy(msg, aggr_sh.at[dstv], add=True)

    enqueue(0, 0)
    enqueue(1, 1)

    def pair_body(k, carry):
        j0 = k * 2
        process(j0, 0)

        @pl.when(j0 + 2 < nk)
        def _():
            enqueue(j0 + 2, 0)

        process(j0 + 1, 1)

        @pl.when(j0 + 3 < nk)
        def _():
            enqueue(j0 + 3, 1)

        return carry

    lax.fori_loop(0, _BASE_K // 2, pair_body, 0)

    @pl.when(nk > _BASE_K)
    def _():
        process(_BASE_K, 0)

    plsc.subcore_barrier()
    pltpu.sync_copy(aggr_sh.at[pl.ds(sid * _NPS, _NPS)],
                    out_hbm.at[pl.ds(cid * _NP + sid * _NPS, _NPS)])


_sc_edge_call = functools.partial(
    pl.kernel,
    out_type=jax.ShapeDtypeStruct((2 * _NP, _CONV), jnp.float32),
    mesh=plsc.VectorSubcoreMesh(core_axis_name="c", subcore_axis_name="s"),
    compiler_params=pltpu.CompilerParams(use_tc_tiling_on_sc=False),
    scratch_types=[
        pltpu.VMEM((_STAGE,), jnp.int32),
        pltpu.VMEM((_SC_CHUNK,), jnp.int32),
        pltpu.VMEM((_SC_CHUNK,), jnp.int32),
        pltpu.VMEM((_SC_CHUNK * 16,), jnp.float32),
        pltpu.VMEM((_SC_CHUNK * 16,), jnp.float32),
        pltpu.VMEM((_SC_CHUNK, _S * _CONV), jnp.float32),
        pltpu.VMEM((_SC_CHUNK, _S * _CONV), jnp.float32),
        pltpu.VMEM((_SC_CHUNK, _CONV), jnp.float32),
        pltpu.VMEM_SHARED((_NP, _CONV), jnp.float32),
        pltpu.SemaphoreType.DMA,
        pltpu.SemaphoreType.DMA,
    ],
)(_sc_edge_body)


def kernel(x, edge_index, C_prime,
           l0_w1, l0_b1, l0_w2, l0_b2, l0_w3, l0_b3, l0_w4, l0_b4, l0_W,
           l0_w5, l0_b5, l0_w6, l0_b6,
           l1_w1, l1_b1, l1_w2, l1_b2, l1_w3, l1_b3, l1_w4, l1_b4, l1_W,
           l1_w5, l1_b5, l1_w6, l1_b6,
           ro_w, ro_b):
    ei = edge_index.astype(jnp.int32)
    src = jnp.pad(ei[0], (0, _SC_CHUNK))
    dst = jnp.pad(ei[1], (0, _SC_CHUNK))
    cpT = C_prime.T
    xp = jnp.pad(x, ((0, _NP - _N), (0, 0)))

    ct0T, ct1T = _ct_call(
        cpT,
        (l0_w1, l0_b1, l0_w2, l0_b2, l0_w3, l0_b3, l0_w4, l0_b4),
        (l1_w1, l1_b1, l1_w2, l1_b2, l1_w3, l1_b3, l1_w4, l1_b4))
    ct0 = jnp.pad(ct0T, ((0, 8), (0, _SC_CHUNK))).T.reshape(-1)
    ct1 = jnp.pad(ct1T, ((0, 8), (0, _SC_CHUNK))).T.reshape(-1)

    W0f = l0_W.transpose(1, 0, 2).reshape(-1, _S * _CONV)
    W1f = l1_W.transpose(1, 0, 2).reshape(-1, _S * _CONV)
    zeros = jnp.zeros((_NP, _CONV), jnp.float32)

    xw1, nm1 = _dense_call(xp, W0f, l0_w5.T, l0_b5, l0_w6.T, l0_b6)
    part1 = _sc_edge_call(xw1, ct0, src, dst, zeros)
    xw2, nm2 = _comb_dense_call(part1, nm1, W1f, l1_w5.T, l1_b5, l1_w6.T, l1_b6)
    part2 = _sc_edge_call(xw2, ct1, src, dst, zeros)
    out = _comb_ro_call(part2, nm2, ro_w.T, ro_b)
    return out[:_N]


# speedup vs baseline: 4.6672x; 1.0346x over previous
import functools

import jax
import jax.numpy as jnp
from jax import lax
from jax.experimental import pallas as pl
from jax.experimental.pallas import tpu as pltpu
from jax.experimental.pallas import tpu_sc as plsc

_N = 10000
_NP = 10240
_E = 160000
_S = 8
_CONV = 64
_MUL = 64
_OUT = 128
_HID = _CONV + _MUL


_BE = 6400


def _ct_body(cp_ref,
             w1a_ref, b1a_ref, w2a_ref, b2a_ref, w3a_ref, b3a_ref,
             w4aa_ref, w4ba_ref, b4a_ref,
             w1b_ref, b1b_ref, w2b_ref, b2b_ref, w3b_ref, b3b_ref,
             w4ab_ref, w4bb_ref, b4b_ref,
             ct0_ref, ct1_ref):
    cp = cp_ref[...]

    def one(w1, b1, w2, b2, w3, b3, w4a, w4b, b4):
        o1 = jax.nn.sigmoid(jnp.dot(w1, cp, preferred_element_type=jnp.float32) + b1)
        o2 = jax.nn.sigmoid(jnp.dot(w2, cp, preferred_element_type=jnp.float32) + b2)
        o3 = jax.nn.sigmoid(jnp.dot(w3, cp, preferred_element_type=jnp.float32) + b3)
        g = o2 * o3
        pre = (jnp.dot(w4a, o1, preferred_element_type=jnp.float32)
               + jnp.dot(w4b, g, preferred_element_type=jnp.float32) + b4)
        return jnp.maximum(pre, 0.0)

    ct0_ref[...] = one(w1a_ref[...], b1a_ref[...], w2a_ref[...], b2a_ref[...],
                       w3a_ref[...], b3a_ref[...], w4aa_ref[...], w4ba_ref[...],
                       b4a_ref[...])
    ct1_ref[...] = one(w1b_ref[...], b1b_ref[...], w2b_ref[...], b2b_ref[...],
                       w3b_ref[...], b3b_ref[...], w4ab_ref[...], w4bb_ref[...],
                       b4b_ref[...])


def _ct_call(cpT, p0, p1):
    wspec = lambda r, c: pl.BlockSpec((r, c), lambda i: (0, 0))
    espec = pl.BlockSpec((_S, _BE), lambda i: (0, i))
    args = []
    for p in (p0, p1):
        (w1, b1, w2, b2, w3, b3, w4, b4) = p
        args += [w1, b1.reshape(_S, 1), w2, b2.reshape(_S, 1),
                 w3, b3.reshape(_S, 1), w4[:, :_S], w4[:, _S:],
                 b4.reshape(_S, 1)]
    in_specs = [espec]
    for _ in range(2):
        in_specs += [wspec(_S, _S), wspec(_S, 1), wspec(_S, _S), wspec(_S, 1),
                     wspec(_S, _S), wspec(_S, 1), wspec(_S, _S), wspec(_S, _S),
                     wspec(_S, 1)]
    return pl.pallas_call(
        _ct_body,
        grid=(_E // _BE,),
        in_specs=in_specs,
        out_specs=[espec, espec],
        out_shape=[jax.ShapeDtypeStruct((_S, _E), jnp.float32)] * 2,
    )(cpT, *args)


_BN = 2048


def _dense_body(x_ref, wf_ref, w5_ref, b5_ref, w6_ref, b6_ref, xw_ref, nm_ref):
    x = x_ref[...]
    xw_ref[...] = jnp.dot(x, wf_ref[...], preferred_element_type=jnp.float32)
    m5 = jnp.dot(x, w5_ref[...], preferred_element_type=jnp.float32) + b5_ref[...]
    m6 = jnp.dot(x, w6_ref[...], preferred_element_type=jnp.float32) + b6_ref[...]
    nm_ref[...] = m5 * m6


def _dense_call(x, Wf, w5T, b5, w6T, b6):
    cin = x.shape[1]
    return pl.pallas_call(
        _dense_body,
        grid=(_NP // _BN,),
        in_specs=[
            pl.BlockSpec((_BN, cin), lambda i: (i, 0)),
            pl.BlockSpec((cin, _S * _CONV), lambda i: (0, 0)),
            pl.BlockSpec((cin, _MUL), lambda i: (0, 0)),
            pl.BlockSpec((1, _MUL), lambda i: (0, 0)),
            pl.BlockSpec((cin, _MUL), lambda i: (0, 0)),
            pl.BlockSpec((1, _MUL), lambda i: (0, 0)),
        ],
        out_specs=[
            pl.BlockSpec((_BN, _S * _CONV), lambda i: (i, 0)),
            pl.BlockSpec((_BN, _MUL), lambda i: (i, 0)),
        ],
        out_shape=[
            jax.ShapeDtypeStruct((_NP, _S * _CONV), jnp.float32),
            jax.ShapeDtypeStruct((_NP, _MUL), jnp.float32),
        ],
    )(x, Wf, w5T, b5.reshape(1, _MUL), w6T, b6.reshape(1, _MUL))


def _comb_dense_body(p0_ref, p1_ref, nm_ref, wf_ref, w5_ref, b5_ref,
                     w6_ref, b6_ref, xw_ref, nm2_ref):
    aggr = jnp.maximum(p0_ref[...] + p1_ref[...], 0.0)
    nm = jnp.maximum(nm_ref[...], 0.0)
    h = jnp.concatenate([aggr, nm], axis=1)
    xw_ref[...] = jnp.dot(h, wf_ref[...], preferred_element_type=jnp.float32)
    m5 = jnp.dot(h, w5_ref[...], preferred_element_type=jnp.float32) + b5_ref[...]
    m6 = jnp.dot(h, w6_ref[...], preferred_element_type=jnp.float32) + b6_ref[...]
    nm2_ref[...] = m5 * m6


def _comb_dense_call(partials, nm, Wf, w5T, b5, w6T, b6):
    nblk = _NP // _BN
    return pl.pallas_call(
        _comb_dense_body,
        grid=(nblk,),
        in_specs=[
            pl.BlockSpec((_BN, _CONV), lambda i: (i, 0)),
            pl.BlockSpec((_BN, _CONV), lambda i: (i + nblk, 0)),
            pl.BlockSpec((_BN, _MUL), lambda i: (i, 0)),
            pl.BlockSpec((_HID, _S * _CONV), lambda i: (0, 0)),
            pl.BlockSpec((_HID, _MUL), lambda i: (0, 0)),
            pl.BlockSpec((1, _MUL), lambda i: (0, 0)),
            pl.BlockSpec((_HID, _MUL), lambda i: (0, 0)),
            pl.BlockSpec((1, _MUL), lambda i: (0, 0)),
        ],
        out_specs=[
            pl.BlockSpec((_BN, _S * _CONV), lambda i: (i, 0)),
            pl.BlockSpec((_BN, _MUL), lambda i: (i, 0)),
        ],
        out_shape=[
            jax.ShapeDtypeStruct((_NP, _S * _CONV), jnp.float32),
            jax.ShapeDtypeStruct((_NP, _MUL), jnp.float32),
        ],
    )(partials, partials, nm, Wf, w5T, b5.reshape(1, _MUL), w6T,
      b6.reshape(1, _MUL))


def _comb_ro_body(p0_ref, p1_ref, nm_ref, ro_ref, rob_ref, out_ref):
    aggr = jnp.maximum(p0_ref[...] + p1_ref[...], 0.0)
    nm = jnp.maximum(nm_ref[...], 0.0)
    h = jnp.concatenate([aggr, nm], axis=1)
    out_ref[...] = (jnp.dot(h, ro_ref[...], preferred_element_type=jnp.float32)
                    + rob_ref[...])


def _comb_ro_call(partials, nm, roT, rob):
    nblk = _NP // _BN
    return pl.pallas_call(
        _comb_ro_body,
        grid=(nblk,),
        in_specs=[
            pl.BlockSpec((_BN, _CONV), lambda i: (i, 0)),
            pl.BlockSpec((_BN, _CONV), lambda i: (i + nblk, 0)),
            pl.BlockSpec((_BN, _MUL), lambda i: (i, 0)),
            pl.BlockSpec((_HID, _OUT), lambda i: (0, 0)),
            pl.BlockSpec((1, _OUT), lambda i: (0, 0)),
        ],
        out_specs=pl.BlockSpec((_BN, _OUT), lambda i: (i, 0)),
        out_shape=jax.ShapeDtypeStruct((_NP, _OUT), jnp.float32),
    )(partials, partials, nm, roT, rob.reshape(1, _OUT))


_NW = 32
_SC_CHUNK = 64
_NCHUNK = _E // _SC_CHUNK
_BASE_K = _NCHUNK // _NW
_EXTRA = _NCHUNK - _BASE_K * _NW
_STAGE = (_BASE_K + 1) * _SC_CHUNK
_NPS = _NP // 16


def _bcast_lane(v, s):
    idx = jnp.full((16,), s, jnp.int32)
    dnums = lax.GatherDimensionNumbers(
        offset_dims=(), collapsed_slice_dims=(0,), start_index_map=(0,))
    return lax.gather(v, idx[:, None], dnums, (1,),
                      mode=lax.GatherScatterMode.PROMISE_IN_BOUNDS)


def _sc_edge_body(xw_hbm, ct_hbm, src_hbm, dst2_hbm, zeros_hbm, out_hbm,
                  src_all, dst_all, ctv0, ctv1, rows0, rows1, msg0, msg1,
                  aggr_sh, sem0, sem1, ssem0, ssem1):
    cid = lax.axis_index("c")
    sid = lax.axis_index("s")
    wid = sid * 2 + cid
    nk = jnp.where(wid < _EXTRA, _BASE_K + 1, _BASE_K)
    base_c = wid * _BASE_K + jnp.minimum(wid, _EXTRA)
    base_e = base_c * _SC_CHUNK

    pltpu.sync_copy(zeros_hbm.at[pl.ds(sid * _NPS, _NPS)],
                    aggr_sh.at[pl.ds(sid * _NPS, _NPS)])
    pltpu.sync_copy(src_hbm.at[pl.ds(base_e, _STAGE)], src_all)
    pltpu.sync_copy(dst2_hbm.at[pl.ds(base_c, _BASE_K + 1)], dst_all)
    plsc.subcore_barrier()

    bufs = ((ctv0, rows0, msg0, sem0, ssem0), (ctv1, rows1, msg1, sem1, ssem1))

    def enqueue(j, b):
        ctv, rows, msg, sem, ssem = bufs[b]
        pltpu.async_copy(xw_hbm.at[src_all.at[pl.ds(j * _SC_CHUNK, _SC_CHUNK)]],
                         rows, sem)
        pltpu.async_copy(
            ct_hbm.at[pl.ds((base_e + j * _SC_CHUNK) * 16, _SC_CHUNK * 16)],
            ctv, sem)

    def process(j, b):
        ctv, rows, msg, sem, ssem = bufs[b]
        pltpu.make_async_copy(
            xw_hbm.at[src_all.at[pl.ds(j * _SC_CHUNK, _SC_CHUNK)]],
            rows, sem).wait()
        pltpu.make_async_copy(
            ct_hbm.at[pl.ds((base_e + j * _SC_CHUNK) * 16, _SC_CHUNK * 16)],
            ctv, sem).wait()

        @pl.when(j >= 2)
        def _():
            pltpu.make_async_copy(msg, aggr_sh.at[dst_all.at[j]], ssem).wait()

        def edge_body(e, c2):
            cvec = ctv[pl.ds(e * 16, 16)]
            coef = [_bcast_lane(cvec, s) for s in range(_S)]
            for cb in range(_CONV // 16):
                acc = rows[e, pl.ds(cb * 16, 16)] * coef[0]
                for s in range(1, _S):
                    acc = acc + rows[e, pl.ds(s * _CONV + cb * 16, 16)] * coef[s]
                msg[e, pl.ds(cb * 16, 16)] = acc
            return c2

        lax.fori_loop(0, _SC_CHUNK, edge_body, 0, unroll=4)
        pltpu.async_copy(msg, aggr_sh.at[dst_all.at[j]], ssem, add=True)

    enqueue(0, 0)
    enqueue(1, 1)

    def pair_body(k, carry):
        j0 = k * 2
        process(j0, 0)

        @pl.when(j0 + 2 < nk)
        def _():
            enqueue(j0 + 2, 0)

        process(j0 + 1, 1)

        @pl.when(j0 + 3 < nk)
        def _():
            enqueue(j0 + 3, 1)

        return carry

    lax.fori_loop(0, _BASE_K // 2, pair_body, 0)

    @pl.when(nk > _BASE_K)
    def _():
        process(_BASE_K, 0)

    for b in (0, 1):
        pltpu.make_async_copy(bufs[b][2], aggr_sh.at[dst_all.at[0]],
                              bufs[b][4]).wait()

    plsc.subcore_barrier()
    pltpu.sync_copy(aggr_sh.at[pl.ds(sid * _NPS, _NPS)],
                    out_hbm.at[pl.ds(cid * _NP + sid * _NPS, _NPS)])


_sc_edge_call = functools.partial(
    pl.kernel,
    out_type=jax.ShapeDtypeStruct((2 * _NP, _CONV), jnp.float32),
    mesh=plsc.VectorSubcoreMesh(core_axis_name="c", subcore_axis_name="s"),
    compiler_params=pltpu.CompilerParams(use_tc_tiling_on_sc=False),
    scratch_types=[
        pltpu.VMEM((_STAGE,), jnp.int32),
        pltpu.VMEM((_BASE_K + 1, _SC_CHUNK), jnp.int32),
        pltpu.VMEM((_SC_CHUNK * 16,), jnp.float32),
        pltpu.VMEM((_SC_CHUNK * 16,), jnp.float32),
        pltpu.VMEM((_SC_CHUNK, _S * _CONV), jnp.float32),
        pltpu.VMEM((_SC_CHUNK, _S * _CONV), jnp.float32),
        pltpu.VMEM((_SC_CHUNK, _CONV), jnp.float32),
        pltpu.VMEM((_SC_CHUNK, _CONV), jnp.float32),
        pltpu.VMEM_SHARED((_NP, _CONV), jnp.float32),
        pltpu.SemaphoreType.DMA,
        pltpu.SemaphoreType.DMA,
        pltpu.SemaphoreType.DMA,
        pltpu.SemaphoreType.DMA,
    ],
)(_sc_edge_body)


def kernel(x, edge_index, C_prime,
           l0_w1, l0_b1, l0_w2, l0_b2, l0_w3, l0_b3, l0_w4, l0_b4, l0_W,
           l0_w5, l0_b5, l0_w6, l0_b6,
           l1_w1, l1_b1, l1_w2, l1_b2, l1_w3, l1_b3, l1_w4, l1_b4, l1_W,
           l1_w5, l1_b5, l1_w6, l1_b6,
           ro_w, ro_b):
    ei = edge_index.astype(jnp.int32)
    src = jnp.pad(ei[0], (0, _SC_CHUNK))
    dst = jnp.pad(ei[1], (0, _SC_CHUNK)).reshape(-1, _SC_CHUNK)
    cpT = C_prime.T
    xp = jnp.pad(x, ((0, _NP - _N), (0, 0)))

    ct0T, ct1T = _ct_call(
        cpT,
        (l0_w1, l0_b1, l0_w2, l0_b2, l0_w3, l0_b3, l0_w4, l0_b4),
        (l1_w1, l1_b1, l1_w2, l1_b2, l1_w3, l1_b3, l1_w4, l1_b4))
    ct0 = jnp.pad(ct0T, ((0, 8), (0, _SC_CHUNK))).T.reshape(-1)
    ct1 = jnp.pad(ct1T, ((0, 8), (0, _SC_CHUNK))).T.reshape(-1)

    W0f = l0_W.transpose(1, 0, 2).reshape(-1, _S * _CONV)
    W1f = l1_W.transpose(1, 0, 2).reshape(-1, _S * _CONV)
    zeros = jnp.zeros((_NP, _CONV), jnp.float32)

    xw1, nm1 = _dense_call(xp, W0f, l0_w5.T, l0_b5, l0_w6.T, l0_b6)
    part1 = _sc_edge_call(xw1, ct0, src, dst, zeros)
    xw2, nm2 = _comb_dense_call(part1, nm1, W1f, l1_w5.T, l1_b5, l1_w6.T, l1_b6)
    part2 = _sc_edge_call(xw2, ct1, src, dst, zeros)
    out = _comb_ro_call(part2, nm2, ro_w.T, ro_b)
    return out[:_N]
